# baseline (speedup 1.0000x reference)
import functools

import jax
import jax.numpy as jnp
from jax import lax
from jax.experimental import pallas as pl
from jax.experimental.pallas import tpu as pltpu
from jax.experimental.pallas import tpu_sc as plsc

_NROW = 4096
_NEXP = 64
_RBLK = 4096
_NR = _NROW // _RBLK
_NB = 4
_NCH = 1024
_NSPLIT = 4
_RSUB = _RBLK // _NSPLIT

_NTOK = _NB * _NCH
_NW = 32
_ROWS_W = _NTOK // _NW


def _gate_body(*refs):
    x_refs = refs[:_NSPLIT]
    wm_ref, wn_ref, bm_ref, bn_ref, out_ref, accm, accn = refs[_NSPLIT:]
    r = pl.program_id(1)

    @pl.when(r == 0)
    def _zero():
        accm[...] = jnp.zeros_like(accm)
        accn[...] = jnp.zeros_like(accn)

    dn = (((1,), (0,)), ((), ()))
    am = jnp.zeros_like(accm)
    an = jnp.zeros_like(accn)
    for i in range(_NSPLIT):
        xb = x_refs[i][0].astype(jnp.bfloat16)
        wm = wm_ref[...][:, i * _RSUB:(i + 1) * _RSUB]
        wn = wn_ref[...][:, i * _RSUB:(i + 1) * _RSUB]
        am += lax.dot_general(wm, xb, dn, preferred_element_type=jnp.float32)
        an += lax.dot_general(wn, xb, dn, preferred_element_type=jnp.float32)
    accm[...] += am
    accn[...] += an

    @pl.when(r == _NR - 1)
    def _finish():
        gm = accm[...] + bm_ref[...]
        gn = accn[...] + bn_ref[...]
        g = gm + jax.nn.softplus(gn)
        out_ref[...] = g.T


def _x_spec(i):
    return pl.BlockSpec((1, _RSUB, _NCH),
                        lambda b, r, i=i: (b, r * _NSPLIT + i, 0))


def _gate_logits(x, W_main, b_main, W_noise, b_noise):
    xm = x.transpose(0, 1, 3, 2).reshape(_NB, _NROW, _NCH)
    Wm = W_main.astype(jnp.bfloat16)
    Wn = W_noise.astype(jnp.bfloat16)
    bm = b_main.reshape(_NEXP, 1)
    bn = b_noise.reshape(_NEXP, 1)
    return pl.pallas_call(
        _gate_body,
        grid=(_NB, _NR),
        in_specs=[_x_spec(i) for i in range(_NSPLIT)] + [
            pl.BlockSpec((_NEXP, _RBLK), lambda b, r: (0, r)),
            pl.BlockSpec((_NEXP, _RBLK), lambda b, r: (0, r)),
            pl.BlockSpec((_NEXP, 1), lambda b, r: (0, 0)),
            pl.BlockSpec((_NEXP, 1), lambda b, r: (0, 0)),
        ],
        out_specs=pl.BlockSpec((_NCH, _NEXP), lambda b, r: (b, 0)),
        out_shape=jax.ShapeDtypeStruct((_NTOK, _NEXP), jnp.float32),
        scratch_shapes=[
            pltpu.VMEM((_NEXP, _NCH), jnp.float32),
            pltpu.VMEM((_NEXP, _NCH), jnp.float32),
        ],
    )(*([xm] * _NSPLIT), Wm, Wn, bm, bn)


@functools.partial(
    pl.kernel,
    out_type=jax.ShapeDtypeStruct((_NTOK, _NEXP), jnp.float32),
    mesh=plsc.VectorSubcoreMesh(core_axis_name="c", subcore_axis_name="s"),
    scratch_types=[
        pltpu.VMEM((_ROWS_W, _NEXP), jnp.float32),
        pltpu.VMEM((_ROWS_W, _NEXP), jnp.float32),
    ],
    compiler_params=pltpu.CompilerParams(needs_layout_passes=False),
)
def _sc_top2_softmax(g_hbm, out_hbm, g_v, o_v):
    wid = lax.axis_index("s") * 2 + lax.axis_index("c")
    base = wid * _ROWS_W
    pltpu.sync_copy(g_hbm.at[pl.ds(base, _ROWS_W)], g_v)
    neg = jnp.float32(-jnp.inf)
    lanes = lax.iota(jnp.int32, 16)

    def group(gi, carry):
        rows = gi * 16 + lanes

        def col(e):
            return plsc.load_gather(g_v, [rows, jnp.full((16,), e, jnp.int32)])

        t1 = jnp.full((16,), neg, dtype=jnp.float32)
        t2 = jnp.full((16,), neg, dtype=jnp.float32)
        t3 = jnp.full((16,), neg, dtype=jnp.float32)
        for e in range(_NEXP):
            v = col(e)
            lo2 = jnp.minimum(t2, v)
            lo1 = jnp.minimum(t1, v)
            t3 = jnp.maximum(t3, lo2)
            t2 = jnp.maximum(t2, lo1)
            t1 = jnp.maximum(t1, v)
        tot = jnp.zeros((16,), jnp.float32)
        for e in range(_NEXP):
            v = col(e)
            ex = jnp.where(v > t3, jnp.exp(v - t1), 0.0)
            tot = tot + ex
            plsc.store_scatter(o_v, [rows, jnp.full((16,), e, jnp.int32)], ex)
        rcp = 1.0 / tot
        for e in range(_NEXP):
            p = plsc.load_gather(o_v, [rows, jnp.full((16,), e, jnp.int32)])
            plsc.store_scatter(o_v, [rows, jnp.full((16,), e, jnp.int32)],
                               p * rcp)
        return carry

    lax.fori_loop(0, _ROWS_W // 16, group, jnp.int32(0))
    pltpu.sync_copy(o_v, out_hbm.at[pl.ds(base, _ROWS_W)])


def kernel(x, W_main, b_main, W_noise, b_noise):
    logits = _gate_logits(x, W_main, b_main, W_noise, b_noise)
    return _sc_top2_softmax(logits)


# speedup vs baseline: 1.1717x; 1.1717x over previous
import functools

import jax
import jax.numpy as jnp
from jax import lax
from jax.experimental import pallas as pl
from jax.experimental.pallas import tpu as pltpu
from jax.experimental.pallas import tpu_sc as plsc

_NROW = 4096
_NEXP = 64
_RBLK = 4096
_NR = _NROW // _RBLK
_NB = 4
_NCH = 1024
_NSPLIT = 4
_RSUB = _RBLK // _NSPLIT

_NTOK = _NB * _NCH
_NW = 32
_COLS_W = _NTOK // _NW


def _gate_body(*refs):
    x_refs = refs[:_NSPLIT]
    wm_ref, wn_ref, bm_ref, bn_ref, out_ref, accm, accn = refs[_NSPLIT:]
    r = pl.program_id(1)

    @pl.when(r == 0)
    def _zero():
        accm[...] = jnp.zeros_like(accm)
        accn[...] = jnp.zeros_like(accn)

    dn = (((1,), (0,)), ((), ()))
    am = jnp.zeros_like(accm)
    an = jnp.zeros_like(accn)
    for i in range(_NSPLIT):
        xb = x_refs[i][0].astype(jnp.bfloat16)
        wm = wm_ref[...][:, i * _RSUB:(i + 1) * _RSUB]
        wn = wn_ref[...][:, i * _RSUB:(i + 1) * _RSUB]
        am += lax.dot_general(wm, xb, dn, preferred_element_type=jnp.float32)
        an += lax.dot_general(wn, xb, dn, preferred_element_type=jnp.float32)
    accm[...] += am
    accn[...] += an

    @pl.when(r == _NR - 1)
    def _finish():
        gm = accm[...] + bm_ref[...]
        gn = accn[...] + bn_ref[...]
        out_ref[...] = gm + jax.nn.softplus(gn)


def _x_spec(i):
    return pl.BlockSpec((1, _RSUB, _NCH),
                        lambda b, r, i=i: (b, r * _NSPLIT + i, 0))


def _gate_logits(x, W_main, b_main, W_noise, b_noise):
    xm = x.transpose(0, 1, 3, 2).reshape(_NB, _NROW, _NCH)
    Wm = W_main.astype(jnp.bfloat16)
    Wn = W_noise.astype(jnp.bfloat16)
    bm = b_main.reshape(_NEXP, 1)
    bn = b_noise.reshape(_NEXP, 1)
    return pl.pallas_call(
        _gate_body,
        grid=(_NB, _NR),
        in_specs=[_x_spec(i) for i in range(_NSPLIT)] + [
            pl.BlockSpec((_NEXP, _RBLK), lambda b, r: (0, r)),
            pl.BlockSpec((_NEXP, _RBLK), lambda b, r: (0, r)),
            pl.BlockSpec((_NEXP, 1), lambda b, r: (0, 0)),
            pl.BlockSpec((_NEXP, 1), lambda b, r: (0, 0)),
        ],
        out_specs=pl.BlockSpec((_NEXP, _NCH), lambda b, r: (0, b)),
        out_shape=jax.ShapeDtypeStruct((_NEXP, _NTOK), jnp.float32),
        scratch_shapes=[
            pltpu.VMEM((_NEXP, _NCH), jnp.float32),
            pltpu.VMEM((_NEXP, _NCH), jnp.float32),
        ],
    )(*([xm] * _NSPLIT), Wm, Wn, bm, bn)


@functools.partial(
    pl.kernel,
    out_type=jax.ShapeDtypeStruct((_NEXP, _NTOK), jnp.float32),
    mesh=plsc.VectorSubcoreMesh(core_axis_name="c", subcore_axis_name="s"),
    scratch_types=[
        pltpu.VMEM((_NEXP, _COLS_W), jnp.float32),
        pltpu.VMEM((_NEXP, _COLS_W), jnp.float32),
    ],
    compiler_params=pltpu.CompilerParams(needs_layout_passes=False),
)
def _sc_top2_softmax(g_hbm, out_hbm, g_v, o_v):
    wid = lax.axis_index("s") * 2 + lax.axis_index("c")
    base = wid * _COLS_W
    pltpu.sync_copy(g_hbm.at[:, pl.ds(base, _COLS_W)], g_v)
    neg = jnp.float32(-jnp.inf)

    def group(j, carry):
        tok = j * 16

        t1 = jnp.full((16,), neg, dtype=jnp.float32)
        t2 = jnp.full((16,), neg, dtype=jnp.float32)
        t3 = jnp.full((16,), neg, dtype=jnp.float32)
        for e in range(_NEXP):
            v = g_v[e, pl.ds(tok, 16)]
            lo2 = jnp.minimum(t2, v)
            lo1 = jnp.minimum(t1, v)
            t3 = jnp.maximum(t3, lo2)
            t2 = jnp.maximum(t2, lo1)
            t1 = jnp.maximum(t1, v)
        tot = jnp.zeros((16,), jnp.float32)
        for e in range(_NEXP):
            v = g_v[e, pl.ds(tok, 16)]
            ex = jnp.where(v > t3, jnp.exp(v - t1), 0.0)
            tot = tot + ex
            o_v[e, pl.ds(tok, 16)] = ex
        rcp = 1.0 / tot
        for e in range(_NEXP):
            o_v[e, pl.ds(tok, 16)] = o_v[e, pl.ds(tok, 16)] * rcp
        return carry

    lax.fori_loop(0, _COLS_W // 16, group, jnp.int32(0))
    pltpu.sync_copy(o_v, out_hbm.at[:, pl.ds(base, _COLS_W)])


def kernel(x, W_main, b_main, W_noise, b_noise):
    logits_t = _gate_logits(x, W_main, b_main, W_noise, b_noise)
    probs_t = _sc_top2_softmax(logits_t)
    return probs_t.T


# speedup vs baseline: 1.2002x; 1.0244x over previous
import functools

import jax
import jax.numpy as jnp
from jax import lax
from jax.experimental import pallas as pl
from jax.experimental.pallas import tpu as pltpu
from jax.experimental.pallas import tpu_sc as plsc

_NROW = 4096
_NEXP = 64
_RBLK = 4096
_NR = _NROW // _RBLK
_NB = 4
_NCH = 1024
_NSPLIT = 4
_RSUB = _RBLK // _NSPLIT

_NTOK = _NB * _NCH
_NW = 32
_COLS_W = _NTOK // _NW


def _gate_body(*refs):
    x_refs = refs[:_NSPLIT]
    wm_ref, wn_ref, bm_ref, bn_ref, out_ref, accm, accn = refs[_NSPLIT:]
    r = pl.program_id(1)

    @pl.when(r == 0)
    def _zero():
        accm[...] = jnp.zeros_like(accm)
        accn[...] = jnp.zeros_like(accn)

    dn = (((1,), (0,)), ((), ()))
    am = jnp.zeros_like(accm)
    an = jnp.zeros_like(accn)
    for i in range(_NSPLIT):
        xb = x_refs[i][0].astype(jnp.bfloat16)
        wm = wm_ref[...][:, i * _RSUB:(i + 1) * _RSUB]
        wn = wn_ref[...][:, i * _RSUB:(i + 1) * _RSUB]
        am += lax.dot_general(wm, xb, dn, preferred_element_type=jnp.float32)
        an += lax.dot_general(wn, xb, dn, preferred_element_type=jnp.float32)
    accm[...] += am
    accn[...] += an

    @pl.when(r == _NR - 1)
    def _finish():
        gm = accm[...] + bm_ref[...]
        gn = accn[...] + bn_ref[...]
        out_ref[...] = gm + jax.nn.softplus(gn)


def _x_spec(i):
    return pl.BlockSpec((1, _RSUB, _NCH),
                        lambda b, r, i=i: (b, r * _NSPLIT + i, 0))


def _gate_logits(x, W_main, b_main, W_noise, b_noise):
    xm = x.transpose(0, 1, 3, 2).reshape(_NB, _NROW, _NCH)
    Wm = W_main.astype(jnp.bfloat16)
    Wn = W_noise.astype(jnp.bfloat16)
    bm = b_main.reshape(_NEXP, 1)
    bn = b_noise.reshape(_NEXP, 1)
    return pl.pallas_call(
        _gate_body,
        grid=(_NB, _NR),
        in_specs=[_x_spec(i) for i in range(_NSPLIT)] + [
            pl.BlockSpec((_NEXP, _RBLK), lambda b, r: (0, r)),
            pl.BlockSpec((_NEXP, _RBLK), lambda b, r: (0, r)),
            pl.BlockSpec((_NEXP, 1), lambda b, r: (0, 0)),
            pl.BlockSpec((_NEXP, 1), lambda b, r: (0, 0)),
        ],
        out_specs=pl.BlockSpec((_NEXP, _NCH), lambda b, r: (0, b)),
        out_shape=jax.ShapeDtypeStruct((_NEXP, _NTOK), jnp.float32),
        scratch_shapes=[
            pltpu.VMEM((_NEXP, _NCH), jnp.float32),
            pltpu.VMEM((_NEXP, _NCH), jnp.float32),
        ],
    )(*([xm] * _NSPLIT), Wm, Wn, bm, bn)


@functools.partial(
    pl.kernel,
    out_type=jax.ShapeDtypeStruct((_NEXP, _NTOK), jnp.float32),
    mesh=plsc.VectorSubcoreMesh(core_axis_name="c", subcore_axis_name="s"),
    scratch_types=[
        pltpu.VMEM((_NEXP, _COLS_W), jnp.float32),
        pltpu.VMEM((_NEXP, _COLS_W), jnp.float32),
    ],
    compiler_params=pltpu.CompilerParams(needs_layout_passes=False),
)
def _sc_top2_softmax(g_hbm, out_hbm, g_v, o_v):
    wid = lax.axis_index("s") * 2 + lax.axis_index("c")
    base = wid * _COLS_W
    pltpu.sync_copy(g_hbm.at[:, pl.ds(base, _COLS_W)], g_v)
    neg = jnp.float32(-jnp.inf)

    def group(j, carry):
        tok = j * 16

        t1 = jnp.full((16,), neg, dtype=jnp.float32)
        t2 = jnp.full((16,), neg, dtype=jnp.float32)
        t3 = jnp.full((16,), neg, dtype=jnp.float32)
        for e in range(_NEXP):
            v = g_v[e, pl.ds(tok, 16)]
            lo2 = jnp.minimum(t2, v)
            lo1 = jnp.minimum(t1, v)
            t3 = jnp.maximum(t3, lo2)
            t2 = jnp.maximum(t2, lo1)
            t1 = jnp.maximum(t1, v)
        tot = jnp.zeros((16,), jnp.float32)
        for e in range(_NEXP):
            v = g_v[e, pl.ds(tok, 16)]
            ex = jnp.where(v > t3, jnp.exp(v - t1), 0.0)
            tot = tot + ex
            o_v[e, pl.ds(tok, 16)] = ex
        rcp = 1.0 / tot
        for e in range(_NEXP):
            o_v[e, pl.ds(tok, 16)] = o_v[e, pl.ds(tok, 16)] * rcp
        return carry

    for j in range(_COLS_W // 16):
        group(j, jnp.int32(0))
    pltpu.sync_copy(o_v, out_hbm.at[:, pl.ds(base, _COLS_W)])


def kernel(x, W_main, b_main, W_noise, b_noise):
    logits_t = _gate_logits(x, W_main, b_main, W_noise, b_noise)
    probs_t = _sc_top2_softmax(logits_t)
    return probs_t.T
